# baseline (speedup 1.0000x reference)
import functools

import jax
import jax.numpy as jnp
from jax import lax
from jax.experimental import pallas as pl
from jax.experimental.pallas import tpu as pltpu
from jax.experimental.pallas import tpu_sc as plsc

D = 16
B = 16384
NF = 16

_info = plsc.get_sparse_core_info()
NC, NS = _info.num_cores, _info.num_subcores
NW = NC * NS
CHUNK = B // NW
SUB = 128
NSUB = CHUNK // SUB


def _sc_kernel(idx_hbm, *rest):
    tabs = rest[:NF]
    out_hbm = rest[NF]
    idx_v, rows_v, sem = rest[NF + 1:]

    wid = lax.axis_index("s") * NC + lax.axis_index("c")
    base = wid * CHUNK
    cbase = wid * NSUB

    pltpu.sync_copy(idx_hbm.at[:, pl.ds(cbase, NSUB), :], idx_v)

    for f in range(NF):
        copies = [
            pltpu.async_copy(
                tabs[f].at[idx_v.at[f, j]],
                rows_v.at[pl.ds(j * SUB, SUB)],
                sem,
            )
            for j in range(NSUB)
        ]
        for c in copies:
            c.wait()
        pltpu.sync_copy(rows_v, out_hbm.at[pl.ds(base, CHUNK), f])


@jax.jit
def _run(xt, *tabs):
    mesh = plsc.VectorSubcoreMesh(core_axis_name="c", subcore_axis_name="s")
    k = functools.partial(
        pl.kernel,
        mesh=mesh,
        out_type=jax.ShapeDtypeStruct((B, NF, D), jnp.float32),
        scratch_types=[
            pltpu.VMEM((NF, NSUB, SUB), jnp.int32),
            pltpu.VMEM((CHUNK, D), jnp.float32),
            pltpu.SemaphoreType.DMA,
        ],
        compiler_params=pltpu.CompilerParams(use_tc_tiling_on_sc=False),
    )(_sc_kernel)
    return k(xt, *tabs)


def kernel(x, table_0, table_1, table_2, table_3, table_4, table_5, table_6,
           table_7, table_8, table_9, table_10, table_11, table_12, table_13,
           table_14, table_15):
    xt = x.T.reshape(NF, B // SUB, SUB)
    return _run(xt, table_0, table_1, table_2, table_3, table_4, table_5,
                table_6, table_7, table_8, table_9, table_10, table_11,
                table_12, table_13, table_14, table_15)


# speedup vs baseline: 1.4000x; 1.4000x over previous
import functools

import jax
import jax.numpy as jnp
from jax import lax
from jax.experimental import pallas as pl
from jax.experimental.pallas import tpu as pltpu
from jax.experimental.pallas import tpu_sc as plsc

D = 16
B = 16384
NF = 16
TSIZES = [1000001, 100001, 102, 3, 32, 13, 4, 5, 45001, 31, 51, 251, 5, 5,
          113, 300001]
BIG = (0, 1, 8, 15)
SMALL = tuple(f for f in range(NF) if f not in BIG)
_SOFF = {}
_off = 0
for _f in SMALL:
    _SOFF[_f] = _off
    _off += TSIZES[_f]
SMALL_W = 640

_info = plsc.get_sparse_core_info()
NC, NS = _info.num_cores, _info.num_subcores
NW = NC * NS
CHUNK = B // NW
SUB = 128
NSUB = CHUNK // SUB

TCW = 2048


def _tc_repack(in_ref, out_ref):
    x = in_ref[...]
    xt3 = x.T.reshape(TCW // 8, 8, 16)
    out_ref[...] = jnp.concatenate([xt3[:, s, :] for s in range(8)], axis=1)


def _repack(tabT):
    s = tabT.shape[1]
    g = (s + TCW - 1) // TCW
    return pl.pallas_call(
        _tc_repack,
        grid=(g,),
        in_specs=[pl.BlockSpec((16, TCW), lambda i: (0, i))],
        out_specs=pl.BlockSpec((TCW // 8, 128), lambda i: (i, 0)),
        out_shape=jax.ShapeDtypeStruct((g * TCW // 8, 128), jnp.float32),
    )(tabT)


def _sc_kernel(xT, smallcat, *rest):
    bigs = rest[:4]
    res = rest[4]
    idx3, jb3, staged, gbufa, gbufb, buf, sem = rest[5:]

    wid = lax.axis_index("s") * NC + lax.axis_index("c")
    base = wid * CHUNK
    iota = lax.iota(jnp.int32, 16)

    for j in range(NSUB):
        pltpu.sync_copy(xT.at[:, pl.ds(base + j * SUB, SUB)], idx3.at[:, j])
    pltpu.sync_copy(smallcat, staged)

    def jbody(v, carry):
        jj = v >> 3
        g16 = (v & 7) * 16
        for bi, f in enumerate(BIG):
            jb3[bi, jj, pl.ds(g16, 16)] = idx3[f, jj, pl.ds(g16, 16)] >> 3
        return carry

    lax.fori_loop(0, NSUB * 8, jbody, 0)

    def extract(f, j, gbuf):
        def ebody(g, carry):
            b0 = j * SUB + g * 16
            rvec = idx3[f, j, pl.ds(g * 16, 16)]
            colbase = (rvec & 7) * 16
            rowvec = g * 16 + iota
            for d in range(D):
                buf[d, pl.ds(b0, 16)] = plsc.load_gather(
                    gbuf, [rowvec, colbase + d])
            return carry

        lax.fori_loop(0, SUB // 16, ebody, 0)

    for bi, f in enumerate(BIG):
        tab = bigs[bi]
        gbufs = (gbufa, gbufb)
        copies = [None, None]
        copies[0] = pltpu.async_copy(tab.at[jb3.at[bi, 0]], gbufa, sem)
        for j in range(NSUB):
            if j + 1 < NSUB:
                copies[(j + 1) % 2] = pltpu.async_copy(
                    tab.at[jb3.at[bi, j + 1]], gbufs[(j + 1) % 2], sem)
            copies[j % 2].wait()
            extract(f, j, gbufs[j % 2])
        pltpu.sync_copy(buf, res.at[f, :, pl.ds(base, CHUNK)])

    for f in SMALL:
        off = _SOFF[f]

        def sbody(g, carry, f=f, off=off):
            rv = idx3[f, g >> 3, pl.ds((g & 7) * 16, 16)] + off
            for d in range(D):
                dv = jnp.full((16,), d, jnp.int32)
                buf[d, pl.ds(g * 16, 16)] = plsc.load_gather(staged, [dv, rv])
            return carry

        lax.fori_loop(0, CHUNK // 16, sbody, 0)
        pltpu.sync_copy(buf, res.at[f, :, pl.ds(base, CHUNK)])


def _run_sc(xT, smallcat, *bigs):
    mesh = plsc.VectorSubcoreMesh(core_axis_name="c", subcore_axis_name="s")
    k = functools.partial(
        pl.kernel,
        mesh=mesh,
        out_type=jax.ShapeDtypeStruct((NF, D, B), jnp.float32),
        scratch_types=[
            pltpu.VMEM((NF, NSUB, SUB), jnp.int32),
            pltpu.VMEM((len(BIG), NSUB, SUB), jnp.int32),
            pltpu.VMEM((D, SMALL_W), jnp.float32),
            pltpu.VMEM((SUB, SUB), jnp.float32),
            pltpu.VMEM((SUB, SUB), jnp.float32),
            pltpu.VMEM((D, CHUNK), jnp.float32),
            pltpu.SemaphoreType.DMA,
        ],
        compiler_params=pltpu.CompilerParams(needs_layout_passes=False),
    )(_sc_kernel)
    return k(xT, smallcat, *bigs)


def kernel(x, table_0, table_1, table_2, table_3, table_4, table_5, table_6,
           table_7, table_8, table_9, table_10, table_11, table_12, table_13,
           table_14, table_15):
    tabs = [table_0, table_1, table_2, table_3, table_4, table_5, table_6,
            table_7, table_8, table_9, table_10, table_11, table_12, table_13,
            table_14, table_15]
    smallcat = jnp.concatenate([tabs[f].T for f in SMALL], axis=1)
    smallcat = jnp.pad(smallcat, ((0, 0), (0, SMALL_W - smallcat.shape[1])))
    bigs = [_repack(tabs[f].T) for f in BIG]
    res = _run_sc(x.T, smallcat, *bigs)
    return res.transpose(2, 0, 1)


# speedup vs baseline: 1.7494x; 1.2496x over previous
import functools

import jax
import jax.numpy as jnp
from jax import lax
from jax.experimental import pallas as pl
from jax.experimental.pallas import tpu as pltpu
from jax.experimental.pallas import tpu_sc as plsc

D = 16
B = 16384
NF = 16
TSIZES = [1000001, 100001, 102, 3, 32, 13, 4, 5, 45001, 31, 51, 251, 5, 5,
          113, 300001]
BIG = (0, 1, 8, 15)
SMALL = tuple(f for f in range(NF) if f not in BIG)
_SOFF = {}
_off = 0
for _f in SMALL:
    _SOFF[_f] = _off
    _off += TSIZES[_f]
SMALL_W = 640

_info = plsc.get_sparse_core_info()
NC, NS = _info.num_cores, _info.num_subcores
NW = NC * NS
CHUNK = B // NW
SUB = 128
NSUB = CHUNK // SUB

TCW = 8192


def _tc_repack(in_ref, out_ref):
    x = in_ref[...]
    xt3 = x.T.reshape(TCW // 8, 8, 16)
    out_ref[...] = jnp.concatenate([xt3[:, s, :] for s in range(8)], axis=1)


def _repack(tabT):
    s = tabT.shape[1]
    g = (s + TCW - 1) // TCW
    return pl.pallas_call(
        _tc_repack,
        grid=(g,),
        in_specs=[pl.BlockSpec((16, TCW), lambda i: (0, i))],
        out_specs=pl.BlockSpec((TCW // 8, 128), lambda i: (i, 0)),
        out_shape=jax.ShapeDtypeStruct((g * TCW // 8, 128), jnp.float32),
    )(tabT)


def _sc_kernel(xT, smallcat, *rest):
    bigs = rest[:4]
    res = rest[4]
    idx3, jb3, staged, gbufa, gbufb, buf, sem = rest[5:]

    wid = lax.axis_index("s") * NC + lax.axis_index("c")
    base = wid * CHUNK
    iota = lax.iota(jnp.int32, 16)

    for j in range(NSUB):
        pltpu.sync_copy(xT.at[:, pl.ds(base + j * SUB, SUB)], idx3.at[:, j])
    pltpu.sync_copy(smallcat, staged)

    def jbody(v, carry):
        jj = v >> 3
        g16 = (v & 7) * 16
        for bi, f in enumerate(BIG):
            jb3[bi, jj, pl.ds(g16, 16)] = idx3[f, jj, pl.ds(g16, 16)] >> 3
        return carry

    lax.fori_loop(0, NSUB * 8, jbody, 0)

    def extract(f, j, gbuf):
        def ebody(g, carry):
            b0 = j * SUB + g * 16
            rvec = idx3[f, j, pl.ds(g * 16, 16)]
            colbase = (rvec & 7) * 16
            rowvec = g * 16 + iota
            for d in range(D):
                buf[d, pl.ds(b0, 16)] = plsc.load_gather(
                    gbuf, [rowvec, colbase + d])
            return carry

        lax.fori_loop(0, SUB // 16, ebody, 0)

    for bi, f in enumerate(BIG):
        tab = bigs[bi]
        gbufs = (gbufa, gbufb)
        copies = [None, None]
        copies[0] = pltpu.async_copy(tab.at[jb3.at[bi, 0]], gbufa, sem)
        for j in range(NSUB):
            if j + 1 < NSUB:
                copies[(j + 1) % 2] = pltpu.async_copy(
                    tab.at[jb3.at[bi, j + 1]], gbufs[(j + 1) % 2], sem)
            copies[j % 2].wait()
            extract(f, j, gbufs[j % 2])
        pltpu.sync_copy(buf, res.at[f, :, pl.ds(base, CHUNK)])

    for f in SMALL:
        off = _SOFF[f]

        def sbody(g, carry, f=f, off=off):
            rv = idx3[f, g >> 3, pl.ds((g & 7) * 16, 16)] + off
            for d in range(D):
                dv = jnp.full((16,), d, jnp.int32)
                buf[d, pl.ds(g * 16, 16)] = plsc.load_gather(staged, [dv, rv])
            return carry

        lax.fori_loop(0, CHUNK // 16, sbody, 0)
        pltpu.sync_copy(buf, res.at[f, :, pl.ds(base, CHUNK)])


def _run_sc(xT, smallcat, *bigs):
    mesh = plsc.VectorSubcoreMesh(core_axis_name="c", subcore_axis_name="s")
    k = functools.partial(
        pl.kernel,
        mesh=mesh,
        out_type=jax.ShapeDtypeStruct((NF, D, B), jnp.float32),
        scratch_types=[
            pltpu.VMEM((NF, NSUB, SUB), jnp.int32),
            pltpu.VMEM((len(BIG), NSUB, SUB), jnp.int32),
            pltpu.VMEM((D, SMALL_W), jnp.float32),
            pltpu.VMEM((SUB, SUB), jnp.float32),
            pltpu.VMEM((SUB, SUB), jnp.float32),
            pltpu.VMEM((D, CHUNK), jnp.float32),
            pltpu.SemaphoreType.DMA,
        ],
        compiler_params=pltpu.CompilerParams(needs_layout_passes=False),
    )(_sc_kernel)
    return k(xT, smallcat, *bigs)


def kernel(x, table_0, table_1, table_2, table_3, table_4, table_5, table_6,
           table_7, table_8, table_9, table_10, table_11, table_12, table_13,
           table_14, table_15):
    tabs = [table_0, table_1, table_2, table_3, table_4, table_5, table_6,
            table_7, table_8, table_9, table_10, table_11, table_12, table_13,
            table_14, table_15]
    smallcat = jnp.concatenate([tabs[f].T for f in SMALL], axis=1)
    smallcat = jnp.pad(smallcat, ((0, 0), (0, SMALL_W - smallcat.shape[1])))
    bigs = [_repack(tabs[f].T) for f in BIG]
    res = _run_sc(x.T, smallcat, *bigs)
    return res.transpose(2, 0, 1)
